# baseline (speedup 1.0000x reference)
import jax
import jax.numpy as jnp
from jax.experimental import pallas as pl
from jax.experimental.pallas import tpu as pltpu

_B = 16
_M = 255
_S = 3
_SHAPES = [(256, 4096), (512, 1024), (1024, 256)]


def _body(x0, x1, x2, w0, b0, w1, b1, w2, b2, o0, o1, o2, *scr):
    (xb0, ob0, wb0, bb0, si0, so0,
     xb1, ob1, wb1, bb1, si1, so1,
     xb2, ob2, wb2, bb2, si2, so2, swb) = scr

    wcps = [
        pltpu.make_async_copy(w0, wb0, swb.at[0]),
        pltpu.make_async_copy(b0, bb0, swb.at[1]),
        pltpu.make_async_copy(w1, wb1, swb.at[2]),
        pltpu.make_async_copy(b1, bb1, swb.at[3]),
        pltpu.make_async_copy(w2, wb2, swb.at[4]),
        pltpu.make_async_copy(b2, bb2, swb.at[5]),
    ]
    for cp in wcps:
        cp.start()
    for cp in wcps:
        cp.wait()

    def run_scale(x_hbm, o_hbm, xb, ob, wb, bb, sin, sout):
        def in_cp(i):
            return pltpu.make_async_copy(
                x_hbm.at[i], xb.at[i % _S], sin.at[i % _S])

        def out_cp(i):
            return pltpu.make_async_copy(
                ob.at[i % _S], o_hbm.at[i], sout.at[i % _S])

        in_cp(0).start(priority=0)
        in_cp(1).start(priority=1)
        for i in range(_B):
            slot = i % _S
            in_cp(i).wait()
            if i + 2 < _B:
                in_cp(i + 2).start(priority=i % 2)
            if i >= _S:
                out_cp(i - _S).wait()
            acc = jnp.dot(wb[...], xb[slot],
                          preferred_element_type=jnp.float32)
            ob[slot] = acc + bb[...]
            out_cp(i).start(priority=(i + 1) % 2)
        for i in range(_B - _S, _B):
            out_cp(i).wait()

    run_scale(x0, o0, xb0, ob0, wb0, bb0, si0, so0)
    run_scale(x1, o1, xb1, ob1, wb1, bb1, si1, so1)
    run_scale(x2, o2, xb2, ob2, wb2, bb2, si2, so2)


def _make_scratch():
    scratch = []
    for c, hw in _SHAPES:
        scratch.append(pltpu.VMEM((_S, c, hw), jnp.float32))
        scratch.append(pltpu.VMEM((_S, _M, hw), jnp.float32))
        scratch.append(pltpu.VMEM((_M, c), jnp.float32))
        scratch.append(pltpu.VMEM((_M, 1), jnp.float32))
        scratch.append(pltpu.SemaphoreType.DMA((_S,)))
        scratch.append(pltpu.SemaphoreType.DMA((_S,)))
    scratch.append(pltpu.SemaphoreType.DMA((6,)))
    return scratch


def kernel(feat0, feat1, feat2, W0, b0, W1, b1, W2, b2):
    xs = [f.reshape(_B, c, hw) for f, (c, hw) in
          zip((feat0, feat1, feat2), _SHAPES)]
    ws = [W.reshape(_M, c) for W, (c, _) in zip((W0, W1, W2), _SHAPES)]
    bs = [b.reshape(_M, 1) for b in (b0, b1, b2)]
    o0, o1, o2 = pl.pallas_call(
        _body,
        in_specs=[pl.BlockSpec(memory_space=pl.ANY)] * 9,
        out_specs=[pl.BlockSpec(memory_space=pl.ANY)] * 3,
        out_shape=tuple(
            jax.ShapeDtypeStruct((_B, _M, hw), jnp.float32)
            for _, hw in _SHAPES
        ),
        scratch_shapes=tuple(_make_scratch()),
    )(xs[0], xs[1], xs[2], ws[0], bs[0], ws[1], bs[1], ws[2], bs[2])
    return (
        o0.reshape(_B, _M, 64, 64),
        o1.reshape(_B, _M, 32, 32),
        o2.reshape(_B, _M, 16, 16),
    )
